# baseline (speedup 1.0000x reference)
import functools

import jax
import jax.numpy as jnp
from jax import lax
from jax.experimental import pallas as pl
from jax.experimental.pallas import tpu as pltpu
from jax.experimental.pallas import tpu_sc as plsc

N = 100000
E = 6400000
B = 16
DT = 30.0

NC = 2
NS = 16
NW = NC * NS
L = 16

EW = E // NW
CE = 800
CE4 = CE // 4
NCH = EW // CE

NP = NW * 3136
NBV = 3136 // L


def _sc_body(src_h, dst_h, flow_h, inw_h, outw_h, nin_h, npr_h, bnd_h,
             batch_h, scal_h, zer_h, out_h,
             tbl, srcbA, dstbA, flwbA, inwbA, outwbA,
             srcbB, dstbB, flwbB, inwbB, outwbB,
             ninb, nprb, bndb, accb, scalb, semA, semB):
    wid = lax.axis_index("s") * NC + lax.axis_index("c")

    setA = (srcbA, dstbA, flwbA, inwbA, outwbA)
    setB = (srcbB, dstbB, flwbB, inwbB, outwbB)

    def refs(k):
        base = pl.multiple_of(wid * EW + k * CE, 8)
        wb = pl.multiple_of(wid * (EW // 4) + k * CE4, 8)
        return (src_h.at[pl.ds(base, CE)],
                dst_h.at[pl.ds(base, CE)],
                flow_h.at[pl.ds(base, CE)],
                inw_h.at[pl.ds(wb, CE4)],
                outw_h.at[pl.ds(wb, CE4)])

    def issue(k, bufs, sem):
        for h, b in zip(refs(k), bufs):
            pltpu.async_copy(h, b, sem)

    def drain(bufs, sem):
        for h, b in zip(refs(0), bufs):
            pltpu.make_async_copy(h, b, sem).wait()

    issue(0, setA, semA)
    issue(1, setB, semB)

    pltpu.sync_copy(batch_h, tbl)
    pltpu.sync_copy(scal_h, scalb)
    pltpu.sync_copy(zer_h, accb)

    estd = scalb[pl.ds(0, L)]
    emean = scalb[pl.ds(L, L)]
    nstd = scalb[pl.ds(2 * L, L)]
    iotav = lax.iota(jnp.int32, L)
    lane16 = iotav * B
    idiv4 = lax.shift_right_logical(iotav, 2)
    shf = (iotav & 3) * 8
    zero16 = jnp.zeros((L,), jnp.int32)

    nbase = wid * 3136
    pltpu.sync_copy(nin_h.at[pl.ds(nbase, 3136)], ninb)
    pltpu.sync_copy(npr_h.at[pl.ds(nbase, 3136)], nprb)
    pltpu.sync_copy(bnd_h.at[pl.ds(nbase, 3136)], bndb)

    def nvec_body(iv):
        sl = pl.ds(iv * L, L)
        bv = tbl[pl.ds(nbase + iv * L, L)]
        v = (nprb[sl] - ninb[sl]) * nstd * (1.0 - bndb[sl])
        plsc.addupdate_scatter(accb, [lane16 + bv], v)

    plsc.parallel_loop(0, NBV, 1, unroll=4)(nvec_body)

    def consume(bufs):
        srcb, dstb, flwb, inwb, outwb = bufs

        def vec_body(iv):
            sl = pl.ds(iv * L, L)
            flw = flwb[sl] * estd + emean
            g1 = plsc.load_gather(tbl, [srcb[sl]])
            g2 = plsc.load_gather(tbl, [dstb[sl]])
            widx = iv * 4 + idiv4
            min_w = plsc.load_gather(inwb, [widx])
            mout_w = plsc.load_gather(outwb, [widx])
            min_b = lax.shift_right_logical(min_w, shf) & 1
            mout_b = lax.shift_right_logical(mout_w, shf) & 1
            plsc.addupdate_scatter(accb, [lane16 + g1], flw * (-DT),
                                   mask=min_b > 0)
            plsc.addupdate_scatter(accb, [lane16 + g2], flw * DT,
                                   mask=mout_b > 0)

        plsc.parallel_loop(0, CE // L, 1, unroll=5)(vec_body)

    def pair_body(j, carry):
        k0 = 2 * j
        drain(setA, semA)
        consume(setA)

        @pl.when(k0 + 2 < NCH)
        def _():
            issue(k0 + 2, setA, semA)

        drain(setB, semB)
        consume(setB)

        @pl.when(k0 + 3 < NCH)
        def _():
            issue(k0 + 3, setB, semB)

        return carry

    lax.fori_loop(0, NCH // 2, pair_body, 0)

    pltpu.sync_copy(accb, out_h.at[wid])


def _tc_body(parts_ref, rain_ref, o_ref):
    s = jnp.sum(parts_ref[...], axis=0, keepdims=True)
    err = s - rain_ref[...]
    o_ref[...] = jnp.sum(jnp.abs(err), axis=1, keepdims=True) * (1.0 / B)


def _pack_mask(m):
    m32 = m.astype(jnp.int32)
    return (m32[0::4] | (m32[1::4] << 8) | (m32[2::4] << 16)
            | (m32[3::4] << 24))


def kernel(batch_node_pred, batch_node_input, batch_edge_input, total_rainfall,
           batch, edge_index, boundary_nodes_mask, inflow_edges_mask,
           outflow_edges_mask, node_mean, node_std, edge_mean, edge_std):
    f32 = jnp.float32
    src = edge_index[0].astype(jnp.int32)
    dst = edge_index[1].astype(jnp.int32)
    inw = _pack_mask(inflow_edges_mask)
    outw = _pack_mask(outflow_edges_mask)
    pad = NP - N
    nin = jnp.pad(batch_node_input[:, 0], (0, pad))
    npr = jnp.pad(batch_node_pred[:, 0], (0, pad))
    bnd = jnp.pad(boundary_nodes_mask.astype(f32), (0, pad),
                  constant_values=1.0)
    batchp = jnp.pad(batch.astype(jnp.int32), (0, pad))
    scal = jnp.concatenate([jnp.full((L,), edge_std, f32),
                            jnp.full((L,), edge_mean, f32),
                            jnp.full((L,), node_std, f32)])
    zer = jnp.zeros((L * B,), f32)

    mesh = plsc.VectorSubcoreMesh(core_axis_name="c", subcore_axis_name="s",
                                  num_cores=NC, num_subcores=NS)
    parts = pl.kernel(
        _sc_body,
        out_type=jax.ShapeDtypeStruct((NW, L * B), f32),
        mesh=mesh,
        compiler_params=pltpu.CompilerParams(needs_layout_passes=False),
        scratch_types=(
            [pltpu.VMEM((NP,), jnp.int32)]
            + 2 * [pltpu.VMEM((CE,), jnp.int32),
                   pltpu.VMEM((CE,), jnp.int32),
                   pltpu.VMEM((CE,), f32),
                   pltpu.VMEM((CE4,), jnp.int32),
                   pltpu.VMEM((CE4,), jnp.int32)]
            + [pltpu.VMEM((3136,), f32),
               pltpu.VMEM((3136,), f32),
               pltpu.VMEM((3136,), f32),
               pltpu.VMEM((L * B,), f32),
               pltpu.VMEM((3 * L,), f32),
               pltpu.SemaphoreType.DMA,
               pltpu.SemaphoreType.DMA]
        ),
    )(src, dst, batch_edge_input[:, 0], inw, outw,
      nin, npr, bnd, batchp, scal, zer)

    loss = pl.pallas_call(
        _tc_body,
        out_shape=jax.ShapeDtypeStruct((1, 1), f32),
    )(parts.reshape(NW * L, B), total_rainfall.reshape(1, B))
    return loss[0, 0]


# speedup vs baseline: 9.3687x; 9.3687x over previous
import functools

import jax
import jax.numpy as jnp
from jax import lax
from jax.experimental import pallas as pl
from jax.experimental.pallas import tpu as pltpu
from jax.experimental.pallas import tpu_sc as plsc

N = 100000
E = 6400000
B = 16
DT = 30.0

NC = 2
NS = 16
NW = NC * NS
L = 16

EW = E // NW
CE = 2000
NCH = EW // CE

NP = NW * 3136
NBV = 3136 // L


def _sc_body(src_h, dst_h, flow_h, inw_h, outw_h, nin_h, npr_h, bnd_h,
             batch_h, scal_h, zer_h, out_h,
             tbl, srcbA, dstbA, flwbA, inwbA, outwbA,
             srcbB, dstbB, flwbB, inwbB, outwbB,
             ninb, nprb, bndb, accb, scalb, semA, semB):
    wid = lax.axis_index("s") * NC + lax.axis_index("c")

    setA = (srcbA, dstbA, flwbA, inwbA, outwbA)
    setB = (srcbB, dstbB, flwbB, inwbB, outwbB)

    def refs(k):
        base = pl.multiple_of(wid * EW + k * CE, 8)
        return (src_h.at[pl.ds(base, CE)],
                dst_h.at[pl.ds(base, CE)],
                flow_h.at[pl.ds(base, CE)],
                inw_h.at[pl.ds(base, CE)],
                outw_h.at[pl.ds(base, CE)])

    def issue(k, bufs, sem):
        for h, b in zip(refs(k), bufs):
            pltpu.async_copy(h, b, sem)

    def drain(bufs, sem):
        for h, b in zip(refs(0), bufs):
            pltpu.make_async_copy(h, b, sem).wait()

    issue(0, setA, semA)
    issue(1, setB, semB)

    pltpu.sync_copy(batch_h, tbl)
    pltpu.sync_copy(scal_h, scalb)
    pltpu.sync_copy(zer_h, accb)

    estd = scalb[pl.ds(0, L)]
    emean = scalb[pl.ds(L, L)]
    nstd = scalb[pl.ds(2 * L, L)]
    iotav = lax.iota(jnp.int32, L)
    lane16 = iotav * B
    idiv4 = lax.shift_right_logical(iotav, 2)
    shf = (iotav & 3) * 8
    zero16 = jnp.zeros((L,), jnp.int32)

    nbase = wid * 3136
    pltpu.sync_copy(nin_h.at[pl.ds(nbase, 3136)], ninb)
    pltpu.sync_copy(npr_h.at[pl.ds(nbase, 3136)], nprb)
    pltpu.sync_copy(bnd_h.at[pl.ds(nbase, 3136)], bndb)

    def nvec_body(iv):
        sl = pl.ds(iv * L, L)
        bv = tbl[pl.ds(nbase + iv * L, L)]
        v = (nprb[sl] - ninb[sl]) * nstd * (1.0 - bndb[sl])
        plsc.addupdate_scatter(accb, [lane16 + bv], v)

    plsc.parallel_loop(0, NBV, 1, unroll=4)(nvec_body)

    def consume(bufs):
        srcb, dstb, flwb, inwb, outwb = bufs

        def vec_body(iv):
            sl = pl.ds(iv * L, L)
            flw = flwb[sl] * estd + emean
            g1 = plsc.load_gather(tbl, [srcb[sl]])
            g2 = plsc.load_gather(tbl, [dstb[sl]])
            plsc.addupdate_scatter(accb, [lane16 + g1], flw * (-DT),
                                   mask=inwb[sl] > 0)
            plsc.addupdate_scatter(accb, [lane16 + g2], flw * DT,
                                   mask=outwb[sl] > 0)

        plsc.parallel_loop(0, CE // L, 1, unroll=5)(vec_body)

    def pair_body(j, carry):
        k0 = 2 * j
        drain(setA, semA)
        consume(setA)

        @pl.when(k0 + 2 < NCH)
        def _():
            issue(k0 + 2, setA, semA)

        drain(setB, semB)
        consume(setB)

        @pl.when(k0 + 3 < NCH)
        def _():
            issue(k0 + 3, setB, semB)

        return carry

    lax.fori_loop(0, NCH // 2, pair_body, 0)

    pltpu.sync_copy(accb, out_h.at[wid])


def _tc_body(parts_ref, rain_ref, o_ref):
    s = jnp.sum(parts_ref[...], axis=0, keepdims=True)
    err = s - rain_ref[...]
    o_ref[...] = jnp.sum(jnp.abs(err), axis=1, keepdims=True) * (1.0 / B)


def _pack_mask(m):
    return m.astype(jnp.int32)


def kernel(batch_node_pred, batch_node_input, batch_edge_input, total_rainfall,
           batch, edge_index, boundary_nodes_mask, inflow_edges_mask,
           outflow_edges_mask, node_mean, node_std, edge_mean, edge_std):
    f32 = jnp.float32
    src = edge_index[0].astype(jnp.int32)
    dst = edge_index[1].astype(jnp.int32)
    inw = _pack_mask(inflow_edges_mask)
    outw = _pack_mask(outflow_edges_mask)
    pad = NP - N
    nin = jnp.pad(batch_node_input[:, 0], (0, pad))
    npr = jnp.pad(batch_node_pred[:, 0], (0, pad))
    bnd = jnp.pad(boundary_nodes_mask.astype(f32), (0, pad),
                  constant_values=1.0)
    batchp = jnp.pad(batch.astype(jnp.int32), (0, pad))
    scal = jnp.concatenate([jnp.full((L,), edge_std, f32),
                            jnp.full((L,), edge_mean, f32),
                            jnp.full((L,), node_std, f32)])
    zer = jnp.zeros((L * B,), f32)

    mesh = plsc.VectorSubcoreMesh(core_axis_name="c", subcore_axis_name="s",
                                  num_cores=NC, num_subcores=NS)
    parts = pl.kernel(
        _sc_body,
        out_type=jax.ShapeDtypeStruct((NW, L * B), f32),
        mesh=mesh,
        compiler_params=pltpu.CompilerParams(needs_layout_passes=False),
        scratch_types=(
            [pltpu.VMEM((NP,), jnp.int32)]
            + 2 * [pltpu.VMEM((CE,), jnp.int32),
                   pltpu.VMEM((CE,), jnp.int32),
                   pltpu.VMEM((CE,), f32),
                   pltpu.VMEM((CE,), jnp.int32),
                   pltpu.VMEM((CE,), jnp.int32)]
            + [pltpu.VMEM((3136,), f32),
               pltpu.VMEM((3136,), f32),
               pltpu.VMEM((3136,), f32),
               pltpu.VMEM((L * B,), f32),
               pltpu.VMEM((3 * L,), f32),
               pltpu.SemaphoreType.DMA,
               pltpu.SemaphoreType.DMA]
        ),
    )(src, dst, batch_edge_input[:, 0], inw, outw,
      nin, npr, bnd, batchp, scal, zer)

    loss = pl.pallas_call(
        _tc_body,
        out_shape=jax.ShapeDtypeStruct((1, 1), f32),
    )(parts.reshape(NW * L, B), total_rainfall.reshape(1, B))
    return loss[0, 0]


# speedup vs baseline: 10.1460x; 1.0830x over previous
import functools

import jax
import jax.numpy as jnp
from jax import lax
from jax.experimental import pallas as pl
from jax.experimental.pallas import tpu as pltpu
from jax.experimental.pallas import tpu_sc as plsc

N = 100000
E = 6400000
B = 16
DT = 30.0

NC = 2
NS = 16
NW = NC * NS
L = 16

EW = E // NW
CE = 2000
NCH = EW // CE

NP = NW * 3136
NBV = 3136 // L


def _sc_body(ei_h, flow_h, inw_h, outw_h, nin_h, npr_h, bnd_h,
             batch_h, scal_h, zer_h, out_h,
             tbl, eibA, flwbA, inwbA, outwbA,
             eibB, flwbB, inwbB, outwbB,
             ninb, nprb, bndb, accb, scalb, semA, semB):
    wid = lax.axis_index("s") * NC + lax.axis_index("c")

    setA = (eibA, flwbA, inwbA, outwbA)
    setB = (eibB, flwbB, inwbB, outwbB)

    def refs(k):
        base = pl.multiple_of(wid * EW + k * CE, 8)
        return (ei_h.at[pl.ds(0, 2), pl.ds(base, CE)],
                flow_h.at[pl.ds(base, CE)],
                inw_h.at[pl.ds(base, CE)],
                outw_h.at[pl.ds(base, CE)])

    def issue(k, bufs, sem):
        for h, b in zip(refs(k), bufs):
            pltpu.async_copy(h, b, sem)

    def drain(bufs, sem):
        for h, b in zip(refs(0), bufs):
            pltpu.make_async_copy(h, b, sem).wait()

    issue(0, setA, semA)
    issue(1, setB, semB)

    pltpu.sync_copy(batch_h, tbl)
    pltpu.sync_copy(scal_h, scalb)
    pltpu.sync_copy(zer_h, accb)

    estd = scalb[pl.ds(0, L)]
    emean = scalb[pl.ds(L, L)]
    nstd = scalb[pl.ds(2 * L, L)]
    iotav = lax.iota(jnp.int32, L)
    lane16 = iotav * B
    zero16 = jnp.zeros((L,), jnp.int32)
    one16 = jnp.ones((L,), jnp.int32)

    nbase = wid * 3136
    pltpu.sync_copy(nin_h.at[pl.ds(nbase, 3136)], ninb)
    pltpu.sync_copy(npr_h.at[pl.ds(nbase, 3136)], nprb)
    pltpu.sync_copy(bnd_h.at[pl.ds(nbase, 3136)], bndb)

    def nvec_body(iv):
        sl = pl.ds(iv * L, L)
        bv = tbl[pl.ds(nbase + iv * L, L)]
        v = (nprb[sl] - ninb[sl]) * nstd * (1.0 - bndb[sl])
        plsc.addupdate_scatter(accb, [lane16 + bv], v)

    plsc.parallel_loop(0, NBV, 1, unroll=4)(nvec_body)

    def consume(bufs):
        eib, flwb, inwb, outwb = bufs

        def vec_body(iv):
            sl = pl.ds(iv * L, L)
            idxv = iv * L + iotav
            flw = flwb[sl] * estd + emean
            g1 = plsc.load_gather(tbl, [plsc.load_gather(eib, [zero16, idxv])])
            g2 = plsc.load_gather(tbl, [plsc.load_gather(eib, [one16, idxv])])
            plsc.addupdate_scatter(accb, [lane16 + g1], flw * (-DT),
                                   mask=inwb[sl] > 0)
            plsc.addupdate_scatter(accb, [lane16 + g2], flw * DT,
                                   mask=outwb[sl] > 0)

        plsc.parallel_loop(0, CE // L, 1, unroll=5)(vec_body)

    def pair_body(j, carry):
        k0 = 2 * j
        drain(setA, semA)
        consume(setA)

        @pl.when(k0 + 2 < NCH)
        def _():
            issue(k0 + 2, setA, semA)

        drain(setB, semB)
        consume(setB)

        @pl.when(k0 + 3 < NCH)
        def _():
            issue(k0 + 3, setB, semB)

        return carry

    lax.fori_loop(0, NCH // 2, pair_body, 0)

    pltpu.sync_copy(accb, out_h.at[wid])


def _tc_body(parts_ref, rain_ref, o_ref):
    s = jnp.sum(parts_ref[...], axis=0, keepdims=True)
    err = s - rain_ref[...]
    o_ref[...] = jnp.sum(jnp.abs(err), axis=1, keepdims=True) * (1.0 / B)


def _pack_mask(m):
    return m.astype(jnp.int32)


def kernel(batch_node_pred, batch_node_input, batch_edge_input, total_rainfall,
           batch, edge_index, boundary_nodes_mask, inflow_edges_mask,
           outflow_edges_mask, node_mean, node_std, edge_mean, edge_std):
    f32 = jnp.float32
    ei = edge_index.astype(jnp.int32)
    inw = _pack_mask(inflow_edges_mask)
    outw = _pack_mask(outflow_edges_mask)
    pad = NP - N
    nin = jnp.pad(batch_node_input[:, 0], (0, pad))
    npr = jnp.pad(batch_node_pred[:, 0], (0, pad))
    bnd = jnp.pad(boundary_nodes_mask.astype(f32), (0, pad),
                  constant_values=1.0)
    batchp = jnp.pad(batch.astype(jnp.int32), (0, pad))
    scal = jnp.concatenate([jnp.full((L,), edge_std, f32),
                            jnp.full((L,), edge_mean, f32),
                            jnp.full((L,), node_std, f32)])
    zer = jnp.zeros((L * B,), f32)

    mesh = plsc.VectorSubcoreMesh(core_axis_name="c", subcore_axis_name="s",
                                  num_cores=NC, num_subcores=NS)
    parts = pl.kernel(
        _sc_body,
        out_type=jax.ShapeDtypeStruct((NW, L * B), f32),
        mesh=mesh,
        compiler_params=pltpu.CompilerParams(needs_layout_passes=False,
                                             use_tc_tiling_on_sc=False),
        scratch_types=(
            [pltpu.VMEM((NP,), jnp.int32)]
            + 2 * [pltpu.VMEM((2, CE), jnp.int32),
                   pltpu.VMEM((CE,), f32),
                   pltpu.VMEM((CE,), jnp.int32),
                   pltpu.VMEM((CE,), jnp.int32)]
            + [pltpu.VMEM((3136,), f32),
               pltpu.VMEM((3136,), f32),
               pltpu.VMEM((3136,), f32),
               pltpu.VMEM((L * B,), f32),
               pltpu.VMEM((3 * L,), f32),
               pltpu.SemaphoreType.DMA,
               pltpu.SemaphoreType.DMA]
        ),
    )(ei, batch_edge_input[:, 0], inw, outw,
      nin, npr, bnd, batchp, scal, zer)

    loss = pl.pallas_call(
        _tc_body,
        out_shape=jax.ShapeDtypeStruct((1, 1), f32),
    )(parts.reshape(NW * L, B), total_rainfall.reshape(1, B))
    return loss[0, 0]
